# baseline (speedup 1.0000x reference)
import functools
import jax
import jax.numpy as jnp
from jax import lax
from jax.experimental import pallas as pl
from jax.experimental.pallas import tpu as pltpu
from jax.experimental.pallas import tpu_sc as plsc

VOCAB = 1000
MAXLEN = 512
WIDTH = VOCAB + MAXLEN
CB = 24
NBANDS = WIDTH // CB
NW = 32
NBUF = 4


def _sc_body(xt_hbm, out_hbm, *scr):
    xrs = scr[0:NBUF]
    bufs = scr[NBUF:2 * NBUF]
    sems = scr[2 * NBUF:3 * NBUF]
    s_len, b = xt_hbm.shape
    nt = s_len * NBANDS
    base, rem = nt // NW, nt % NW
    wid = lax.axis_index("s") * 2 + lax.axis_index("c")
    t0 = wid * base + jnp.minimum(wid, rem)
    cnt = base + jnp.where(wid < rem, 1, 0)
    kmax = base + (1 if rem else 0)
    ones16 = jnp.full((16,), 1.0, jnp.float32)
    zeros16 = jnp.zeros((16,), jnp.float32)

    def zero_buf(buf):
        def zr(r, _):
            def zg(g, _):
                buf[r, pl.ds(g * 16, 16)] = zeros16
                return 0
            return lax.fori_loop(0, b // 16, zg, 0)
        lax.fori_loop(0, CB, zr, 0)

    for buf in bufs:
        zero_buf(buf)

    def scatter_band(buf, xr, c0, val):
        def sg(g, _):
            xv = xr[pl.ds(g * 16, 16)]
            msk = (xv >= c0) & (xv < c0 + CB)
            b_idx = lax.broadcasted_iota(jnp.int32, (16,), 0) + g * 16
            row = jnp.where(msk, xv - c0, 0)
            plsc.store_scatter(buf, [row, b_idx], val, mask=msk)
            return 0
        lax.fori_loop(0, b // 16, sg, 0)

    def pos_row(buf, s, c0, val):
        pr = VOCAB + s - c0

        @pl.when((pr >= 0) & (pr < CB))
        def _():
            def pg(g, _):
                buf[pr, pl.ds(g * 16, 16)] = val
                return 0
            lax.fori_loop(0, b // 16, pg, 0)

    def step(k, buf, xr, sem):
        t = t0 + k
        s = t // NBANDS
        c0 = (t - s * NBANDS) * CB

        @pl.when(k >= NBUF)
        def _():
            tp = t - NBUF
            sp = tp // NBANDS
            cp = (tp - sp * NBANDS) * CB
            pltpu.make_async_copy(buf, out_hbm.at[sp, pl.ds(cp, CB)], sem).wait()
            scatter_band(buf, xr, cp, zeros16)
            pos_row(buf, sp, cp, zeros16)

            @pl.when(sp != s)
            def _():
                pltpu.sync_copy(xt_hbm.at[s], xr)

        @pl.when(k < NBUF)
        def _():
            pltpu.sync_copy(xt_hbm.at[s], xr)

        scatter_band(buf, xr, c0, ones16)
        pos_row(buf, s, c0, ones16)
        pltpu.async_copy(buf, out_hbm.at[s, pl.ds(c0, CB)], sem)

    def task(k, _):
        @pl.when(k < cnt)
        def _():
            for p in range(NBUF):
                @pl.when(k % NBUF == p)
                def _(p=p):
                    step(k, bufs[p], xrs[p], sems[p])

        return 0

    lax.fori_loop(0, kmax, task, 0)

    def drain(k, buf, sem):
        t = t0 + k
        s = t // NBANDS
        c0 = (t - s * NBANDS) * CB
        pltpu.make_async_copy(buf, out_hbm.at[s, pl.ds(c0, CB)], sem).wait()

    for j in range(1, NBUF + 1):
        @pl.when(cnt >= j)
        def _(j=j):
            k = cnt - j
            for p in range(NBUF):
                @pl.when(k % NBUF == p)
                def _(p=p, k=k):
                    drain(k, bufs[p], sems[p])


def kernel(x):
    b, s = x.shape
    xt = x.T
    mesh = plsc.VectorSubcoreMesh(core_axis_name="c", subcore_axis_name="s")
    scratch = (
        [pltpu.VMEM((b,), jnp.int32) for _ in range(NBUF)]
        + [pltpu.VMEM((CB, b), jnp.float32) for _ in range(NBUF)]
        + [pltpu.SemaphoreType.DMA for _ in range(NBUF)]
    )
    sck = functools.partial(
        pl.kernel,
        mesh=mesh,
        out_type=jax.ShapeDtypeStruct((s, WIDTH, b), jnp.float32),
        scratch_types=scratch,
        compiler_params=pltpu.CompilerParams(needs_layout_passes=False),
    )(_sc_body)
    out = sck(xt)
    return out.transpose(2, 0, 1)
